# baseline (speedup 1.0000x reference)
import functools

import jax
import jax.numpy as jnp
from jax import lax
from jax.experimental import pallas as pl
from jax.experimental.pallas import tpu as pltpu
from jax.experimental.pallas import tpu_sc as plsc

_EPS = 1e-5
_TOPK = 2
_BLK = 512
_GBLK = 512


def _dot_t(a, b, precision=None):
    return jax.lax.dot_general(
        a, b, dimension_numbers=(((1,), (1,)), ((), ())),
        precision=precision, preferred_element_type=jnp.float32)


def _silu(v):
    return v * jax.nn.sigmoid(v)


def _ln(x_ref, lnw_ref, lnb_ref):
    xb = x_ref[...]
    mu = jnp.mean(xb, axis=-1, keepdims=True)
    xc = xb - mu
    var = jnp.mean(xc * xc, axis=-1, keepdims=True)
    return xc / jnp.sqrt(var + _EPS) * lnw_ref[...] + lnb_ref[...]


def _gates_body(x_ref, lnw_ref, lnb_ref, wcat_ref, bcat_ref, gout_ref):
    flat = _ln(x_ref, lnw_ref, lnb_ref)
    gout_ref[...] = _dot_t(wcat_ref[...], flat) + bcat_ref[...]


def _routing_sc(gcat, G, EPG):
    E = G * EPG
    S = gcat.shape[1]
    NC, NS, L = 2, 16, 16
    NW = NC * NS
    TPW = S // NW
    mesh = plsc.VectorSubcoreMesh(core_axis_name="c", subcore_axis_name="s")

    @functools.partial(
        pl.kernel, mesh=mesh,
        out_type=jax.ShapeDtypeStruct((E, S), jnp.float32),
        scratch_types=[pltpu.VMEM((G + E, TPW), jnp.float32),
                       pltpu.VMEM((E, TPW), jnp.float32)])
    def k(gcat_hbm, c8_hbm, buf, cbuf):
        wid = lax.axis_index("s") * NC + lax.axis_index("c")
        base = wid * TPW
        for r in range(G + E):
            pltpu.sync_copy(gcat_hbm.at[r, pl.ds(base, TPW)], buf.at[r])

        @pl.loop(0, TPW, step=L)
        def _(c):
            sl = pl.ds(c, L)
            g0 = buf[0, sl]
            g1 = buf[1, sl]
            gm = g1 > g0
            ls = []
            for i in range(EPG):
                ls.append(jnp.where(gm, buf[G + EPG + i, sl], buf[G + i, sl]))
            m = ls[0]
            for l in ls[1:]:
                m = jnp.maximum(m, l)
            exs = [jnp.exp(l - m) for l in ls]
            ssum = exs[0]
            for e_ in exs[1:]:
                ssum = ssum + e_
            ps = [e_ / ssum for e_ in exs]
            zero = jnp.zeros_like(ps[0])
            one = zero + 1.0
            for i in range(EPG):
                cnt = zero
                for j in range(EPG):
                    if j == i:
                        continue
                    beats = (ps[j] >= ps[i]) if j < i else (ps[j] > ps[i])
                    cnt = cnt + jnp.where(beats, one, zero)
                ci = jnp.where(cnt < float(_TOPK), ps[i], zero)
                cbuf[i, sl] = jnp.where(gm, zero, ci)
                cbuf[EPG + i, sl] = jnp.where(gm, ci, zero)

        for r in range(E):
            pltpu.sync_copy(cbuf.at[r], c8_hbm.at[r, pl.ds(base, TPW)])

    return k(gcat)


def _dense_core(first, G, EPG, H, x_ref, c8_ref, lnw_ref, lnb_ref, bso_ref,
                beo_ref, wso_ref, wsi_hbm, wei_hbm, weo_hbm, out_ref,
                wsi_v, wei_v, weo_v, sem_si, sem_ei, sem_o1, sem_o2):
    E = G * EPG
    if first:
        pltpu.make_async_copy(wsi_hbm, wsi_v, sem_si).start()
        pltpu.make_async_copy(wei_hbm, wei_v, sem_ei).start()
        pltpu.make_async_copy(weo_hbm.at[:EPG], weo_v.at[:EPG], sem_o1).start()
        pltpu.make_async_copy(weo_hbm.at[EPG:], weo_v.at[EPG:], sem_o2).start()

    flat = _ln(x_ref, lnw_ref, lnb_ref)
    c8 = jnp.transpose(c8_ref[...])

    if first:
        pltpu.make_async_copy(wsi_hbm, wsi_v, sem_si).wait()
    h_shared = (_silu(_dot_t(flat, wsi_v[:H, :])) * _dot_t(flat, wsi_v[H:, :]))
    acc = _dot_t(h_shared, wso_ref[...]) + bso_ref[...]

    if first:
        pltpu.make_async_copy(wei_hbm, wei_v, sem_ei).wait()
    h_expert = (_silu(_dot_t(flat, wei_v[:H, :])) * _dot_t(flat, wei_v[H:, :]))

    if first:
        pltpu.make_async_copy(weo_hbm.at[:EPG], weo_v.at[:EPG], sem_o1).wait()
    for e in range(EPG):
        w = c8[:, e:e + 1]
        acc = acc + w * (_dot_t(h_expert, weo_v[e]) + beo_ref[e:e + 1, :])

    if first:
        pltpu.make_async_copy(weo_hbm.at[EPG:], weo_v.at[EPG:], sem_o2).wait()
    for e in range(EPG, E):
        w = c8[:, e:e + 1]
        acc = acc + w * (_dot_t(h_expert, weo_v[e]) + beo_ref[e:e + 1, :])
    out_ref[...] = acc


def _dense_body(G, EPG, H, *refs):
    i = pl.program_id(0)

    @pl.when(i == 0)
    def _first_step():
        _dense_core(True, G, EPG, H, *refs)

    @pl.when(i != 0)
    def _steady_state():
        _dense_core(False, G, EPG, H, *refs)


def kernel(x, ln_w, ln_b, w_shared_in, w_shared_out, b_shared_out,
           w_expert_in, expert_out_w, expert_out_b,
           w_group_gate, w_expert_gate, group_bias, expert_bias):
    B, T, C = x.shape
    S = B * T
    G = w_group_gate.shape[0]
    E = expert_out_w.shape[0]
    EPG = E // G
    H = w_shared_out.shape[1]
    flat_x = x.reshape(S, C)
    lnw2 = ln_w.reshape(1, C)
    lnb2 = ln_b.reshape(1, C)

    wcat = jnp.concatenate([w_group_gate, w_expert_gate], axis=0)
    bcat = jnp.concatenate([group_bias, expert_bias]).reshape(G + E, 1)

    const2 = lambda i: (0, 0)
    anyspec = pl.BlockSpec(memory_space=pl.ANY)

    gcat = pl.pallas_call(
        _gates_body,
        grid=(S // _GBLK,),
        in_specs=[
            pl.BlockSpec((_GBLK, C), lambda i: (i, 0)),
            pl.BlockSpec((1, C), const2),
            pl.BlockSpec((1, C), const2),
            pl.BlockSpec((G + E, C), const2),
            pl.BlockSpec((G + E, 1), const2),
        ],
        out_specs=pl.BlockSpec((G + E, _GBLK), lambda i: (0, i)),
        out_shape=jax.ShapeDtypeStruct((G + E, S), jnp.float32),
        compiler_params=pltpu.CompilerParams(
            dimension_semantics=("arbitrary",)),
    )(flat_x, lnw2, lnb2, wcat, bcat)

    c8t = _routing_sc(gcat, G, EPG)

    out = pl.pallas_call(
        functools.partial(_dense_body, G, EPG, H),
        grid=(S // _BLK,),
        in_specs=[
            pl.BlockSpec((_BLK, C), lambda i: (i, 0)),
            pl.BlockSpec((E, _BLK), lambda i: (0, i)),
            pl.BlockSpec((1, C), const2),
            pl.BlockSpec((1, C), const2),
            pl.BlockSpec((1, C), const2),
            pl.BlockSpec((E, C), const2),
            pl.BlockSpec((C, H), const2),
            anyspec,
            anyspec,
            anyspec,
        ],
        out_specs=pl.BlockSpec((_BLK, C), lambda i: (i, 0)),
        out_shape=jax.ShapeDtypeStruct((S, C), jnp.float32),
        scratch_shapes=[
            pltpu.VMEM((2 * H, C), jnp.float32),
            pltpu.VMEM((2 * H, C), jnp.float32),
            pltpu.VMEM((E, C, H), jnp.float32),
            pltpu.SemaphoreType.DMA,
            pltpu.SemaphoreType.DMA,
            pltpu.SemaphoreType.DMA,
            pltpu.SemaphoreType.DMA,
        ],
        compiler_params=pltpu.CompilerParams(
            dimension_semantics=("arbitrary",),
            vmem_limit_bytes=128 * 1024 * 1024,
        ),
    )(flat_x, c8t, lnw2, lnb2, b_shared_out.reshape(1, C), expert_out_b,
      w_shared_out, w_shared_in, w_expert_in, expert_out_w)
    return out.reshape(B, T, C)


# speedup vs baseline: 1.0456x; 1.0456x over previous
import functools

import jax
import jax.numpy as jnp
from jax import lax
from jax.experimental import pallas as pl
from jax.experimental.pallas import tpu as pltpu
from jax.experimental.pallas import tpu_sc as plsc

_EPS = 1e-5
_TOPK = 2
_BLK = 512
_GBLK = 512


def _dot_t(a, b, precision=None):
    return jax.lax.dot_general(
        a, b, dimension_numbers=(((1,), (1,)), ((), ())),
        precision=precision, preferred_element_type=jnp.float32)


def _silu(v):
    return v * jax.nn.sigmoid(v)


def _ln(x_ref, lnw_ref, lnb_ref):
    xb = x_ref[...]
    mu = jnp.mean(xb, axis=-1, keepdims=True)
    xc = xb - mu
    var = jnp.mean(xc * xc, axis=-1, keepdims=True)
    return xc / jnp.sqrt(var + _EPS) * lnw_ref[...] + lnb_ref[...]


def _gates_body(x_ref, lnw_ref, lnb_ref, wcat_ref, bcat_ref, gout_ref):
    flat = _ln(x_ref, lnw_ref, lnb_ref)
    gout_ref[...] = _dot_t(wcat_ref[...], flat) + bcat_ref[...]


def _routing_sc(gcat, G, EPG):
    E = G * EPG
    S = gcat.shape[1]
    NC, NS, L = 2, 16, 16
    NWA = 16
    TPW = S // NWA
    mesh = plsc.VectorSubcoreMesh(core_axis_name="c", subcore_axis_name="s")

    @functools.partial(
        pl.kernel, mesh=mesh,
        out_type=jax.ShapeDtypeStruct((E, S), jnp.float32),
        scratch_types=[pltpu.VMEM((G + E, TPW), jnp.float32),
                       pltpu.VMEM((E, TPW), jnp.float32)])
    def k(gcat_hbm, c8_hbm, buf, cbuf):
        wid = lax.axis_index("s") * NC + lax.axis_index("c")

        @pl.when(wid < NWA)
        def _active_worker():
            base = wid * TPW
            pltpu.sync_copy(gcat_hbm.at[:, pl.ds(base, TPW)], buf)

            @pl.loop(0, TPW, step=L)
            def _(c):
                sl = pl.ds(c, L)
                g0 = buf[0, sl]
                g1 = buf[1, sl]
                gm = g1 > g0
                ls = []
                for i in range(EPG):
                    ls.append(
                        jnp.where(gm, buf[G + EPG + i, sl], buf[G + i, sl]))
                m = ls[0]
                for l in ls[1:]:
                    m = jnp.maximum(m, l)
                exs = [jnp.exp(l - m) for l in ls]
                ssum = exs[0]
                for e_ in exs[1:]:
                    ssum = ssum + e_
                ps = [e_ / ssum for e_ in exs]
                zero = jnp.zeros_like(ps[0])
                one = zero + 1.0
                for i in range(EPG):
                    cnt = zero
                    for j in range(EPG):
                        if j == i:
                            continue
                        beats = (ps[j] >= ps[i]) if j < i else (ps[j] > ps[i])
                        cnt = cnt + jnp.where(beats, one, zero)
                    ci = jnp.where(cnt < float(_TOPK), ps[i], zero)
                    cbuf[i, sl] = jnp.where(gm, zero, ci)
                    cbuf[EPG + i, sl] = jnp.where(gm, ci, zero)

            pltpu.sync_copy(cbuf, c8_hbm.at[:, pl.ds(base, TPW)])

    return k(gcat)


def _dense_core(first, G, EPG, H, x_ref, c8_ref, lnw_ref, lnb_ref, bso_ref,
                beo_ref, wso_ref, wsi_hbm, wei_hbm, weo_hbm, out_ref,
                wsi_v, wei_v, weo_v, sem_si, sem_ei, sem_o1, sem_o2):
    E = G * EPG
    if first:
        pltpu.make_async_copy(wsi_hbm, wsi_v, sem_si).start()
        pltpu.make_async_copy(wei_hbm, wei_v, sem_ei).start()
        pltpu.make_async_copy(weo_hbm.at[:EPG], weo_v.at[:EPG], sem_o1).start()
        pltpu.make_async_copy(weo_hbm.at[EPG:], weo_v.at[EPG:], sem_o2).start()

    flat = _ln(x_ref, lnw_ref, lnb_ref)
    c8 = jnp.transpose(c8_ref[...])

    if first:
        pltpu.make_async_copy(wsi_hbm, wsi_v, sem_si).wait()
    h_shared = (_silu(_dot_t(flat, wsi_v[:H, :])) * _dot_t(flat, wsi_v[H:, :]))
    acc = _dot_t(h_shared, wso_ref[...]) + bso_ref[...]

    if first:
        pltpu.make_async_copy(wei_hbm, wei_v, sem_ei).wait()
    h_expert = (_silu(_dot_t(flat, wei_v[:H, :])) * _dot_t(flat, wei_v[H:, :]))

    if first:
        pltpu.make_async_copy(weo_hbm.at[:EPG], weo_v.at[:EPG], sem_o1).wait()
    for e in range(EPG):
        w = c8[:, e:e + 1]
        acc = acc + w * (_dot_t(h_expert, weo_v[e]) + beo_ref[e:e + 1, :])

    if first:
        pltpu.make_async_copy(weo_hbm.at[EPG:], weo_v.at[EPG:], sem_o2).wait()
    for e in range(EPG, E):
        w = c8[:, e:e + 1]
        acc = acc + w * (_dot_t(h_expert, weo_v[e]) + beo_ref[e:e + 1, :])
    out_ref[...] = acc


def _dense_body(G, EPG, H, *refs):
    i = pl.program_id(0)

    @pl.when(i == 0)
    def _first_step():
        _dense_core(True, G, EPG, H, *refs)

    @pl.when(i != 0)
    def _steady_state():
        _dense_core(False, G, EPG, H, *refs)


def kernel(x, ln_w, ln_b, w_shared_in, w_shared_out, b_shared_out,
           w_expert_in, expert_out_w, expert_out_b,
           w_group_gate, w_expert_gate, group_bias, expert_bias):
    B, T, C = x.shape
    S = B * T
    G = w_group_gate.shape[0]
    E = expert_out_w.shape[0]
    EPG = E // G
    H = w_shared_out.shape[1]
    flat_x = x.reshape(S, C)
    lnw2 = ln_w.reshape(1, C)
    lnb2 = ln_b.reshape(1, C)

    wcat = jnp.concatenate([w_group_gate, w_expert_gate], axis=0)
    bcat = jnp.concatenate([group_bias, expert_bias]).reshape(G + E, 1)

    const2 = lambda i: (0, 0)
    anyspec = pl.BlockSpec(memory_space=pl.ANY)

    gcat = pl.pallas_call(
        _gates_body,
        grid=(S // _GBLK,),
        in_specs=[
            pl.BlockSpec((_GBLK, C), lambda i: (i, 0)),
            pl.BlockSpec((1, C), const2),
            pl.BlockSpec((1, C), const2),
            pl.BlockSpec((G + E, C), const2),
            pl.BlockSpec((G + E, 1), const2),
        ],
        out_specs=pl.BlockSpec((G + E, _GBLK), lambda i: (0, i)),
        out_shape=jax.ShapeDtypeStruct((G + E, S), jnp.float32),
        compiler_params=pltpu.CompilerParams(
            dimension_semantics=("arbitrary",)),
    )(flat_x, lnw2, lnb2, wcat, bcat)

    c8t = _routing_sc(gcat, G, EPG)

    out = pl.pallas_call(
        functools.partial(_dense_body, G, EPG, H),
        grid=(S // _BLK,),
        in_specs=[
            pl.BlockSpec((_BLK, C), lambda i: (i, 0)),
            pl.BlockSpec((E, _BLK), lambda i: (0, i)),
            pl.BlockSpec((1, C), const2),
            pl.BlockSpec((1, C), const2),
            pl.BlockSpec((1, C), const2),
            pl.BlockSpec((E, C), const2),
            pl.BlockSpec((C, H), const2),
            anyspec,
            anyspec,
            anyspec,
        ],
        out_specs=pl.BlockSpec((_BLK, C), lambda i: (i, 0)),
        out_shape=jax.ShapeDtypeStruct((S, C), jnp.float32),
        scratch_shapes=[
            pltpu.VMEM((2 * H, C), jnp.float32),
            pltpu.VMEM((2 * H, C), jnp.float32),
            pltpu.VMEM((E, C, H), jnp.float32),
            pltpu.SemaphoreType.DMA,
            pltpu.SemaphoreType.DMA,
            pltpu.SemaphoreType.DMA,
            pltpu.SemaphoreType.DMA,
        ],
        compiler_params=pltpu.CompilerParams(
            dimension_semantics=("arbitrary",),
            vmem_limit_bytes=128 * 1024 * 1024,
        ),
    )(flat_x, c8t, lnw2, lnb2, b_shared_out.reshape(1, C), expert_out_b,
      w_shared_out, w_shared_in, w_expert_in, expert_out_w)
    return out.reshape(B, T, C)
